# baseline (device time: 94419 ns/iter reference)
import jax
import jax.numpy as jnp
from jax import lax
from jax.experimental import pallas as pl
from jax.experimental.pallas import tpu as pltpu

N_DEV = 4
N_TOK = 2048
D_IN = 512
D_OUT = 1024
E_LOCAL = 4
CAPACITY = 102
CHUNK = N_TOK // N_DEV


def kernel(x, router_W, route_idx, expert_W):
    del router_W

    def body(x_ref, idx_ref, w_ref, out_ref,
             keep_ref, partial_ref, comm_ref, send_sems, recv_sems):
        my = lax.axis_index("i")
        left = jnp.mod(my - 1, N_DEV)
        right = jnp.mod(my + 1, N_DEV)

        barrier = pltpu.get_barrier_semaphore()
        for nbr in (left, right):
            pl.semaphore_signal(
                barrier, inc=1,
                device_id=(nbr,), device_id_type=pl.DeviceIdType.MESH,
            )
        pl.semaphore_wait(barrier, 2)

        local_ids = my * E_LOCAL + lax.broadcasted_iota(
            jnp.int32, (1, E_LOCAL), 1
        )
        oh = (idx_ref[:, :] == local_ids).astype(jnp.float32)
        c = oh
        shift = 1
        while shift < N_TOK:
            c = c + jnp.concatenate(
                [jnp.zeros((shift, E_LOCAL), jnp.float32), c[:-shift]], axis=0
            )
            shift *= 2
        keep_ref[:, :] = oh * (c <= float(CAPACITY)).astype(jnp.float32)

        for j in range(N_DEV):
            cj = jnp.mod(my - 1 - j, N_DEV)
            base = cj * CHUNK
            xc = x_ref[pl.ds(base, CHUNK), :]
            kc = keep_ref[pl.ds(base, CHUNK), :]
            acc = jnp.zeros((CHUNK, D_OUT), jnp.float32)
            for k in range(E_LOCAL):
                acc = acc + jnp.dot(
                    xc * kc[:, k][:, None], w_ref[k],
                    preferred_element_type=jnp.float32,
                )
            partial_ref[j] = acc

        for s in range(N_DEV - 1):
            rdma = pltpu.make_async_remote_copy(
                src_ref=partial_ref.at[s],
                dst_ref=comm_ref.at[s],
                send_sem=send_sems.at[s],
                recv_sem=recv_sems.at[s],
                device_id=(right,),
                device_id_type=pl.DeviceIdType.MESH,
            )
            rdma.start()
            rdma.wait()
            partial_ref[s + 1] = partial_ref[s + 1] + comm_ref[s]

        out_ref[:, :] = partial_ref[N_DEV - 1]

    return pl.pallas_call(
        body,
        out_shape=jax.ShapeDtypeStruct((CHUNK, D_OUT), jnp.float32),
        in_specs=[
            pl.BlockSpec(memory_space=pltpu.VMEM),
            pl.BlockSpec(memory_space=pltpu.VMEM),
            pl.BlockSpec(memory_space=pltpu.VMEM),
        ],
        out_specs=pl.BlockSpec(memory_space=pltpu.VMEM),
        scratch_shapes=[
            pltpu.VMEM((N_TOK, E_LOCAL), jnp.float32),
            pltpu.VMEM((N_DEV, CHUNK, D_OUT), jnp.float32),
            pltpu.VMEM((N_DEV - 1, CHUNK, D_OUT), jnp.float32),
            pltpu.SemaphoreType.DMA((N_DEV - 1,)),
            pltpu.SemaphoreType.DMA((N_DEV - 1,)),
        ],
        compiler_params=pltpu.CompilerParams(collective_id=0),
    )(x, route_idx, expert_W)


# device time: 61853 ns/iter; 1.5265x vs baseline; 1.5265x over previous
import jax
import jax.numpy as jnp
from jax import lax
from jax.experimental import pallas as pl
from jax.experimental.pallas import tpu as pltpu

N_DEV = 4
N_TOK = 2048
D_IN = 512
D_OUT = 1024
E_LOCAL = 4
CAPACITY = 102
CHUNK = N_TOK // N_DEV


def kernel(x, router_W, route_idx, expert_W):
    del router_W

    def body(x_ref, idx_ref, w_ref, out_ref,
             keep_ref, send_ref, comm_ref, send_sems, recv_sems):
        my = lax.axis_index("i")

        sends = [(2, 0), (1, 1), (3, 2)]

        barrier = pltpu.get_barrier_semaphore()
        for off in (1, 2, 3):
            pl.semaphore_signal(
                barrier, inc=1,
                device_id=(jnp.mod(my + off, N_DEV),),
                device_id_type=pl.DeviceIdType.MESH,
            )
        pl.semaphore_wait(barrier, 3)

        local_ids = my * E_LOCAL + lax.broadcasted_iota(
            jnp.int32, (1, E_LOCAL), 1
        )
        oh = (idx_ref[:, :] == local_ids).astype(jnp.float32)
        c = oh
        shift = 1
        while shift < N_TOK:
            c = c + jnp.concatenate(
                [jnp.zeros((shift, E_LOCAL), jnp.float32), c[:-shift]], axis=0
            )
            shift *= 2
        keep_ref[:, :] = oh * (c <= float(CAPACITY)).astype(jnp.float32)

        def compute_chunk(chunk_idx):
            base = chunk_idx * CHUNK
            xc = x_ref[pl.ds(base, CHUNK), :]
            kc = keep_ref[pl.ds(base, CHUNK), :]
            acc = jnp.zeros((CHUNK, D_OUT), jnp.float32)
            for k in range(E_LOCAL):
                acc = acc + jnp.dot(
                    xc * kc[:, k][:, None], w_ref[k],
                    preferred_element_type=jnp.float32,
                )
            return acc

        rdmas = []
        for j, (off, slot) in enumerate(sends):
            send_ref[j] = compute_chunk(jnp.mod(my + off, N_DEV))
            rdma = pltpu.make_async_remote_copy(
                src_ref=send_ref.at[j],
                dst_ref=comm_ref.at[slot],
                send_sem=send_sems.at[j],
                recv_sem=recv_sems.at[slot],
                device_id=(jnp.mod(my + off, N_DEV),),
                device_id_type=pl.DeviceIdType.MESH,
            )
            rdma.start()
            rdmas.append(rdma)

        out_ref[:, :] = compute_chunk(my)

        for slot in (1, 2, 0):
            rdmas_by_slot = {s: r for (_, s), r in zip(sends, rdmas)}
            rdmas_by_slot[slot].wait_recv()
            out_ref[:, :] = out_ref[:, :] + comm_ref[slot]
        for rdma in rdmas:
            rdma.wait_send()

    return pl.pallas_call(
        body,
        out_shape=jax.ShapeDtypeStruct((CHUNK, D_OUT), jnp.float32),
        in_specs=[
            pl.BlockSpec(memory_space=pltpu.VMEM),
            pl.BlockSpec(memory_space=pltpu.VMEM),
            pl.BlockSpec(memory_space=pltpu.VMEM),
        ],
        out_specs=pl.BlockSpec(memory_space=pltpu.VMEM),
        scratch_shapes=[
            pltpu.VMEM((N_TOK, E_LOCAL), jnp.float32),
            pltpu.VMEM((N_DEV - 1, CHUNK, D_OUT), jnp.float32),
            pltpu.VMEM((N_DEV - 1, CHUNK, D_OUT), jnp.float32),
            pltpu.SemaphoreType.DMA((N_DEV - 1,)),
            pltpu.SemaphoreType.DMA((N_DEV - 1,)),
        ],
        compiler_params=pltpu.CompilerParams(collective_id=0),
    )(x, route_idx, expert_W)
